# Initial kernel scaffold; baseline (speedup 1.0000x reference)
import jax
import jax.numpy as jnp
from jax.experimental import pallas as pl


def kernel(x, edge_index, edge_attr, W0, b0, W1, b1, W2, b2):
    raise NotImplementedError("write your pallas kernel here")


# speedup vs baseline: 4.5247x; 4.5247x over previous
import functools

import jax
import jax.numpy as jnp
from jax import lax
from jax.experimental import pallas as pl
from jax.experimental.pallas import tpu as pltpu
from jax.experimental.pallas import tpu_sc as plsc

NUM_CORES = 2
NUM_SUBCORES = 16
LANES = 16
CHUNK = 128


def _sc_gather_scale_scatter(h, src, dst, w):
    N, D = h.shape
    E = src.shape[0]
    assert E % CHUNK == 0 and N % NUM_SUBCORES == 0 and D % LANES == 0
    nchunks = E // CHUNK
    nw = NUM_CORES * NUM_SUBCORES
    niter = (nchunks + nw - 1) // nw
    rows_per_tile = (N // (8 * NUM_SUBCORES)) * 8
    leftover = N - rows_per_tile * NUM_SUBCORES
    n_full, rem = divmod(rows_per_tile, CHUNK)
    assert leftover % 8 == 0 and leftover <= CHUNK

    mesh = plsc.VectorSubcoreMesh(
        core_axis_name="c", subcore_axis_name="s",
        num_cores=NUM_CORES, num_subcores=NUM_SUBCORES)

    @functools.partial(
        pl.kernel,
        out_type=jax.ShapeDtypeStruct((NUM_CORES, N, D), jnp.float32),
        mesh=mesh,
        scratch_types=[
            pltpu.VMEM((CHUNK,), jnp.int32),
            pltpu.VMEM((CHUNK,), jnp.int32),
            pltpu.VMEM((CHUNK,), jnp.float32),
            pltpu.VMEM((CHUNK, D), jnp.float32),
            pltpu.VMEM_SHARED((N, D), jnp.float32),
            pltpu.SemaphoreType.DMA,
        ],
    )
    def sck(h_hbm, src_hbm, dst_hbm, w_hbm, out_hbm,
            src_v, dst_v, w_v, rows_v, agg_sh, sem):
        c = lax.axis_index("c")
        s = lax.axis_index("s")
        wid = s * NUM_CORES + c
        iota16 = lax.iota(jnp.int32, LANES)
        zero16 = jnp.zeros((LANES,), jnp.float32)

        def zrow(i, carry):
            for k in range(D // LANES):
                rows_v[i, pl.ds(k * LANES, LANES)] = zero16
            return carry
        lax.fori_loop(0, CHUNK, zrow, 0)

        r0 = s * rows_per_tile
        for j in range(n_full):
            pltpu.sync_copy(rows_v.at[pl.ds(0, CHUNK)],
                            agg_sh.at[pl.ds(r0 + j * CHUNK, CHUNK)])
        if rem:
            pltpu.sync_copy(rows_v.at[pl.ds(0, rem)],
                            agg_sh.at[pl.ds(r0 + n_full * CHUNK, rem)])
        if leftover:
            @pl.when(s == 0)
            def _():
                pltpu.sync_copy(
                    rows_v.at[pl.ds(0, leftover)],
                    agg_sh.at[pl.ds(rows_per_tile * NUM_SUBCORES, leftover)])
        plsc.subcore_barrier()

        def body(j, carry):
            ci = wid + j * nw

            @pl.when(ci < nchunks)
            def _():
                base = ci * CHUNK
                pltpu.sync_copy(src_hbm.at[pl.ds(base, CHUNK)], src_v)
                pltpu.sync_copy(dst_hbm.at[pl.ds(base, CHUNK)], dst_v)
                pltpu.sync_copy(w_hbm.at[pl.ds(base, CHUNK)], w_v)
                pltpu.async_copy(h_hbm.at[src_v], rows_v, sem).wait()

                def egroup(g, inner):
                    wg = w_v[pl.ds(g * LANES, LANES)]
                    for i in range(LANES):
                        e = g * LANES + i
                        wb = jnp.full((LANES,), wg[i])
                        for k in range(D // LANES):
                            sl = pl.ds(k * LANES, LANES)
                            rows_v[e, sl] = rows_v[e, sl] * wb
                    return inner
                lax.fori_loop(0, CHUNK // LANES, egroup, 0)
                pltpu.sync_copy(rows_v, agg_sh.at[dst_v], add=True)
            return carry
        lax.fori_loop(0, niter, body, 0)
        plsc.subcore_barrier()

        for j in range(n_full):
            sl = pl.ds(r0 + j * CHUNK, CHUNK)
            pltpu.sync_copy(agg_sh.at[sl], out_hbm.at[c, sl])
        if rem:
            sl = pl.ds(r0 + n_full * CHUNK, rem)
            pltpu.sync_copy(agg_sh.at[sl], out_hbm.at[c, sl])
        if leftover:
            @pl.when(s == 0)
            def _():
                sl = pl.ds(rows_per_tile * NUM_SUBCORES, leftover)
                pltpu.sync_copy(agg_sh.at[sl], out_hbm.at[c, sl])

    return sck(h, src, dst, w)


def _tc_linear(h, p0, p1, W, b, relu):
    N, D = h.shape
    blk = 1000
    assert N % blk == 0

    def body(h_ref, p0_ref, p1_ref, w_ref, b_ref, o_ref):
        rst = h_ref[...] + p0_ref[...] + p1_ref[...]
        acc = jnp.dot(rst, w_ref[...],
                      preferred_element_type=jnp.float32) + b_ref[...]
        o_ref[...] = jnp.maximum(acc, 0.0) if relu else acc

    return pl.pallas_call(
        body,
        grid=(N // blk,),
        in_specs=[
            pl.BlockSpec((blk, D), lambda i: (i, 0)),
            pl.BlockSpec((blk, D), lambda i: (i, 0)),
            pl.BlockSpec((blk, D), lambda i: (i, 0)),
            pl.BlockSpec((D, D), lambda i: (0, 0)),
            pl.BlockSpec((1, D), lambda i: (0, 0)),
        ],
        out_specs=pl.BlockSpec((blk, D), lambda i: (i, 0)),
        out_shape=jax.ShapeDtypeStruct((N, D), jnp.float32),
    )(h, p0, p1, W, b.reshape(1, D))


def kernel(x, edge_index, edge_attr, W0, b0, W1, b1, W2, b2):
    src = edge_index[0]
    dst = edge_index[1]
    w = edge_attr
    E = src.shape[0]
    if E % CHUNK:
        pad = CHUNK - E % CHUNK
        src = jnp.pad(src, (0, pad))
        dst = jnp.pad(dst, (0, pad))
        w = jnp.pad(w, (0, pad))

    h = x
    for i, (W, b) in enumerate(((W0, b0), (W1, b1), (W2, b2))):
        parts = _sc_gather_scale_scatter(h, src, dst, w)
        h = _tc_linear(h, parts[0], parts[1], W, b, relu=(i < 2))
    return h
